# Initial kernel scaffold; baseline (speedup 1.0000x reference)
import jax
import jax.numpy as jnp
from jax.experimental import pallas as pl


def kernel(idx, features_packed, zbuf):
    raise NotImplementedError("write your pallas kernel here")


# speedup vs baseline: 9.6363x; 9.6363x over previous
import functools

import jax
import jax.numpy as jnp
from jax import lax
from jax.experimental import pallas as pl
from jax.experimental.pallas import tpu as pltpu
from jax.experimental.pallas import tpu_sc as plsc

_B, _H, _W, _K = 4, 512, 512, 1
_P, _C = 200000, 64
_N = _B * _H * _W
_D = _C + 1

_NC, _NS = 2, 16
_NW = _NC * _NS
_PER_W = _N // _NW
_CHUNK = 128
_NCH = _PER_W // _CHUNK


def _sc_feature_gather(safe_idx, table):
    mesh = plsc.VectorSubcoreMesh(core_axis_name="c", subcore_axis_name="s")

    @functools.partial(
        pl.kernel,
        out_type=jax.ShapeDtypeStruct((_N, _D), jnp.float32),
        mesh=mesh,
        scratch_types=[
            pltpu.VMEM((_PER_W,), jnp.int32),
            pltpu.VMEM((_CHUNK, _C), jnp.float32),
            pltpu.VMEM((_CHUNK, _D), jnp.float32),
            pltpu.SemaphoreType.DMA,
        ],
        compiler_params=pltpu.CompilerParams(
            use_tc_tiling_on_sc=False, needs_layout_passes=False
        ),
    )
    def body(idx_hbm, table_hbm, feat_hbm, idx_v, rows_v, pack_v, sem):
        wid = lax.axis_index("s") * _NC + lax.axis_index("c")
        base = pl.multiple_of(wid * _PER_W, 8)
        pltpu.sync_copy(idx_hbm.at[pl.ds(base, _PER_W)], idx_v)

        lane = lax.broadcasted_iota(jnp.int32, (16,), 0)

        def chunk(j, carry):
            off = pl.multiple_of(j * _CHUNK, 8)
            pltpu.async_copy(
                table_hbm.at[idx_v.at[pl.ds(off, _CHUNK)]], rows_v, sem
            ).wait()

            def row(r, c2):
                rvec = jnp.full((16,), r, jnp.int32)
                for t in range(_C // 16):
                    v = rows_v[r, pl.ds(16 * t, 16)]
                    plsc.store_scatter(pack_v, [rvec, lane + 16 * t], v)
                return c2

            lax.fori_loop(0, _CHUNK, row, 0)

            def alpha(a, c2):
                iv = idx_v[pl.ds(off + 16 * a, 16)]
                av = jnp.where(iv < _P, 1.0, 0.0).astype(jnp.float32)
                plsc.store_scatter(
                    pack_v, [lane + 16 * a, jnp.full((16,), _C, jnp.int32)], av
                )
                return c2

            lax.fori_loop(0, _CHUNK // 16, alpha, 0)

            pltpu.sync_copy(pack_v, feat_hbm.at[pl.ds(base + off, _CHUNK)])
            return carry

        lax.fori_loop(0, _NCH, chunk, 0)

    return body(safe_idx, table)


def _tc_depth(idx_flat, zbuf_flat):
    rows, cols = 2048, 512
    blk = 256

    def body(idx_ref, zb_ref, out_ref):
        out_ref[...] = jnp.where(idx_ref[...] >= 0, zb_ref[...], 0.0)

    out = pl.pallas_call(
        body,
        out_shape=jax.ShapeDtypeStruct((rows, cols), jnp.float32),
        grid=(rows // blk,),
        in_specs=[
            pl.BlockSpec((blk, cols), lambda i: (i, 0)),
            pl.BlockSpec((blk, cols), lambda i: (i, 0)),
        ],
        out_specs=pl.BlockSpec((blk, cols), lambda i: (i, 0)),
    )(idx_flat.reshape(rows, cols), zbuf_flat.reshape(rows, cols))
    return out.reshape(-1)


def kernel(idx, features_packed, zbuf):
    idx32 = idx.reshape(_N).astype(jnp.int32)
    safe_idx = jnp.where(idx32 >= 0, idx32, _P)
    table = jnp.concatenate(
        [features_packed, jnp.zeros((1, _C), jnp.float32)], axis=0
    )

    feat = _sc_feature_gather(safe_idx, table)
    depth = _tc_depth(idx32, zbuf.reshape(_N))
    return (
        feat.reshape(_B, _H, _W, _D),
        depth.reshape(_B, _H, _W, _K),
    )


# speedup vs baseline: 11.9481x; 1.2399x over previous
import functools

import jax
import jax.numpy as jnp
from jax import lax
from jax.experimental import pallas as pl
from jax.experimental.pallas import tpu as pltpu
from jax.experimental.pallas import tpu_sc as plsc

_B, _H, _W, _K = 4, 512, 512, 1
_P, _C = 200000, 64
_N = _B * _H * _W
_D = _C + 1

_NC, _NS = 2, 16
_NW = _NC * _NS
_PER_W = _N // _NW
_CHUNK = 128
_NCH = _PER_W // _CHUNK


_NBUF = 4
_OUTER = _NCH // _NBUF


def _sc_feature_gather(safe_idx, table):
    mesh = plsc.VectorSubcoreMesh(core_axis_name="c", subcore_axis_name="s")

    @functools.partial(
        pl.kernel,
        out_type=jax.ShapeDtypeStruct((_N, _D), jnp.float32),
        mesh=mesh,
        scratch_types=[
            pltpu.VMEM((_PER_W,), jnp.int32),
            [pltpu.VMEM((_CHUNK, _C), jnp.float32) for _ in range(_NBUF)],
            [pltpu.VMEM((_CHUNK, _D), jnp.float32) for _ in range(_NBUF)],
            [pltpu.SemaphoreType.DMA for _ in range(_NBUF)],
            [pltpu.SemaphoreType.DMA for _ in range(_NBUF)],
        ],
        compiler_params=pltpu.CompilerParams(
            use_tc_tiling_on_sc=False, needs_layout_passes=False
        ),
    )
    def body(idx_hbm, table_hbm, feat_hbm, idx_v, rows, packs, isems, osems):
        wid = lax.axis_index("s") * _NC + lax.axis_index("c")
        base = pl.multiple_of(wid * _PER_W, 8)
        pltpu.sync_copy(idx_hbm.at[pl.ds(base, _PER_W)], idx_v)

        lane = lax.broadcasted_iota(jnp.int32, (16,), 0)

        def gather_src(g):
            off = pl.multiple_of(g * _CHUNK, 8)
            return table_hbm.at[idx_v.at[pl.ds(off, _CHUNK)]]

        def feat_dst(g):
            return feat_hbm.at[pl.ds(pl.multiple_of(base + g * _CHUNK, 8), _CHUNK)]

        def repack(g, b):
            off = pl.multiple_of(g * _CHUNK, 8)

            def row(r, c2):
                rvec = jnp.full((16,), r, jnp.int32)
                for t in range(_C // 16):
                    v = rows[b][r, pl.ds(16 * t, 16)]
                    plsc.store_scatter(packs[b], [rvec, lane + 16 * t], v)
                return c2

            lax.fori_loop(0, _CHUNK, row, 0)

            def alpha(a, c2):
                iv = idx_v[pl.ds(off + 16 * a, 16)]
                av = jnp.where(iv < _P, 1.0, 0.0).astype(jnp.float32)
                plsc.store_scatter(
                    packs[b], [lane + 16 * a, jnp.full((16,), _C, jnp.int32)], av
                )
                return c2

            lax.fori_loop(0, _CHUNK // 16, alpha, 0)

        for b in range(_NBUF):
            pltpu.async_copy(gather_src(b), rows[b], isems[b])

        def outer(o, carry):
            for b in range(_NBUF):
                g = o * _NBUF + b
                pltpu.make_async_copy(gather_src(g), rows[b], isems[b]).wait()
                @pl.when(o > 0)
                def _():
                    pltpu.make_async_copy(
                        packs[b], feat_dst(g - _NBUF), osems[b]
                    ).wait()

                repack(g, b)
                pltpu.async_copy(packs[b], feat_dst(g), osems[b])

                @pl.when(o < _OUTER - 1)
                def _():
                    pltpu.async_copy(gather_src(g + _NBUF), rows[b], isems[b])
            return carry

        lax.fori_loop(0, _OUTER, outer, 0)

        for b in range(_NBUF):
            g = (_OUTER - 1) * _NBUF + b
            pltpu.make_async_copy(packs[b], feat_dst(g), osems[b]).wait()

    return body(safe_idx, table)


def _tc_depth(idx_flat, zbuf_flat):
    rows, cols = 2048, 512
    blk = 256

    def body(idx_ref, zb_ref, out_ref):
        out_ref[...] = jnp.where(idx_ref[...] >= 0, zb_ref[...], 0.0)

    out = pl.pallas_call(
        body,
        out_shape=jax.ShapeDtypeStruct((rows, cols), jnp.float32),
        grid=(rows // blk,),
        in_specs=[
            pl.BlockSpec((blk, cols), lambda i: (i, 0)),
            pl.BlockSpec((blk, cols), lambda i: (i, 0)),
        ],
        out_specs=pl.BlockSpec((blk, cols), lambda i: (i, 0)),
    )(idx_flat.reshape(rows, cols), zbuf_flat.reshape(rows, cols))
    return out.reshape(-1)


def kernel(idx, features_packed, zbuf):
    idx32 = idx.reshape(_N).astype(jnp.int32)
    safe_idx = jnp.where(idx32 >= 0, idx32, _P)
    table = jnp.concatenate(
        [features_packed, jnp.zeros((1, _C), jnp.float32)], axis=0
    )

    feat = _sc_feature_gather(safe_idx, table)
    depth = _tc_depth(idx32, zbuf.reshape(_N))
    return (
        feat.reshape(_B, _H, _W, _D),
        depth.reshape(_B, _H, _W, _K),
    )


# speedup vs baseline: 16.8737x; 1.4122x over previous
import functools

import jax
import jax.numpy as jnp
from jax import lax
from jax.experimental import pallas as pl
from jax.experimental.pallas import tpu as pltpu
from jax.experimental.pallas import tpu_sc as plsc

_B, _H, _W, _K = 4, 512, 512, 1
_P, _C = 200000, 64
_N = _B * _H * _W
_D = _C + 1

_NC, _NS = 2, 16
_NW = _NC * _NS
_PER_W = _N // _NW
_SUB = 128
_TCH = 32
_QPX = 8192


def _sc_feature_gather(safe_swz, table):
    mesh = plsc.VectorSubcoreMesh(core_axis_name="c", subcore_axis_name="s")

    @functools.partial(
        pl.kernel,
        out_type=jax.ShapeDtypeStruct((_B, _D, _H, _W), jnp.float32),
        mesh=mesh,
        scratch_types=[
            pltpu.VMEM((_QPX,), jnp.int32),
            [pltpu.VMEM((_SUB, 128), jnp.float32) for _ in range(2)],
            pltpu.VMEM((_D, 8, 128), jnp.float32),
            [pltpu.SemaphoreType.DMA for _ in range(2)],
            pltpu.SemaphoreType.DMA,
        ],
        compiler_params=pltpu.CompilerParams(
            use_tc_tiling_on_sc=True, needs_layout_passes=False
        ),
    )
    def body(idx_hbm, table_hbm, out_hbm, idx_q, rows, tbuf, gsems, osem):
        wid = lax.axis_index("s") * _NC + lax.axis_index("c")
        base = pl.multiple_of(wid * _PER_W, 8)
        bat = wid // 8
        ch0 = (wid % 8) * _TCH

        lane = lax.broadcasted_iota(jnp.int32, (16,), 0)

        def load_quarter(q):
            pltpu.sync_copy(
                idx_hbm.at[pl.ds(pl.multiple_of(base + q * _QPX, 8), _QPX)], idx_q
            )

        def start_gather(sg, slot):
            off = pl.multiple_of((sg % 64) * _SUB, 8)
            return pltpu.async_copy(
                table_hbm.at[idx_q.at[pl.ds(off, _SUB)]], rows[slot], gsems[slot]
            )

        def wait_gather(slot):
            pltpu.make_async_copy(
                table_hbm.at[idx_q.at[pl.ds(0, _SUB)]], rows[slot], gsems[slot]
            ).wait()

        def out_dst(tc):
            ch = ch0 + tc
            hb = (ch // 4) % 64
            wb = ch % 4
            return out_hbm.at[
                bat, :, pl.ds(hb * 8, 8), pl.ds(pl.multiple_of(wb * 128, 8), 128)
            ]

        zeros16 = jnp.full((16,), 0, jnp.int32)

        def extract(slot, b):
            for k in range(_SUB // 16):
                rowv = lane + 16 * k

                def cbody(c):
                    colv = zeros16 + c
                    v = plsc.load_gather(rows[slot], [rowv, colv])
                    tbuf[c, b, pl.ds(16 * k, 16)] = v

                plsc.parallel_loop(0, _D, 1, unroll=8)(cbody)

        load_quarter(0)
        start_gather(0, 0)
        start_gather(1, 1)

        def chunk(tc, carry):
            for b in range(8):
                sg = tc * 8 + b
                slot = b % 2
                if b == 0:
                    @pl.when((tc % 8 == 0) & (tc > 0))
                    def _():
                        load_quarter(tc // 8)
                        start_gather(sg, 0)
                        start_gather(sg + 1, 1)

                wait_gather(slot)
                if b == 0:
                    @pl.when(tc > 0)
                    def _():
                        pltpu.make_async_copy(tbuf, out_dst(tc - 1), osem).wait()

                extract(slot, b)

                if b < 6:
                    start_gather(sg + 2, slot)
                else:
                    @pl.when(tc % 8 != 7)
                    def _():
                        start_gather(sg + 2, slot)
            pltpu.async_copy(tbuf, out_dst(tc), osem)
            return carry

        lax.fori_loop(0, _TCH, chunk, 0)
        pltpu.make_async_copy(tbuf, out_dst(_TCH - 1), osem).wait()

    return body(safe_swz, table)


def _tc_depth(idx_flat, zbuf_flat):
    rows, cols = 2048, 512
    blk = 256

    def body(idx_ref, zb_ref, out_ref):
        out_ref[...] = jnp.where(idx_ref[...] >= 0, zb_ref[...], 0.0)

    out = pl.pallas_call(
        body,
        out_shape=jax.ShapeDtypeStruct((rows, cols), jnp.float32),
        grid=(rows // blk,),
        in_specs=[
            pl.BlockSpec((blk, cols), lambda i: (i, 0)),
            pl.BlockSpec((blk, cols), lambda i: (i, 0)),
        ],
        out_specs=pl.BlockSpec((blk, cols), lambda i: (i, 0)),
    )(idx_flat.reshape(rows, cols), zbuf_flat.reshape(rows, cols))
    return out.reshape(-1)


def kernel(idx, features_packed, zbuf):
    idx32 = idx.reshape(_N).astype(jnp.int32)
    safe_idx = jnp.where(idx32 >= 0, idx32, _P)
    safe_swz = (
        safe_idx.reshape(_B, _H // 8, 8, _W // 128, 128)
        .transpose(0, 1, 3, 2, 4)
        .reshape(_N)
    )
    table = jnp.pad(
        jnp.concatenate([features_packed, jnp.ones((_P, 1), jnp.float32)], axis=1),
        ((0, 8), (0, 128 - _D)),
    )

    feat_planar = _sc_feature_gather(safe_swz, table)
    feat = feat_planar.transpose(0, 2, 3, 1)
    depth = _tc_depth(idx32, zbuf.reshape(_N))
    return (
        feat,
        depth.reshape(_B, _H, _W, _K),
    )
